# Initial kernel scaffold; baseline (speedup 1.0000x reference)
import jax
import jax.numpy as jnp
from jax.experimental import pallas as pl


def kernel(xyz1, xyz2, points1, points2, W_fuse, g_fuse, b_fuse, W1, g1, b1, W2, g2, b2):
    raise NotImplementedError("write your pallas kernel here")


# speedup vs baseline: 19.9142x; 19.9142x over previous
import functools
import math

import jax
import jax.numpy as jnp
from jax.experimental import pallas as pl
from jax.experimental.pallas import tpu as pltpu

_INV_SQRT2 = 1.0 / math.sqrt(2.0)


def _gelu(x):
    return 0.5 * x * (1.0 + jax.lax.erf(x * _INV_SQRT2))


def _bn_coeffs(stats_ref, g_ref, b_ref, cnt):
    mean = stats_ref[:, 0:1] / cnt
    var = stats_ref[:, 1:2] / cnt - mean * mean
    scale = g_ref[...] * jax.lax.rsqrt(var + 1e-5)
    off = b_ref[...] - mean * scale
    return scale, off


def _stats_update(t):
    s = jnp.sum(t, axis=1, keepdims=True)
    ss = jnp.sum(t * t, axis=1, keepdims=True)
    z = jnp.zeros((t.shape[0], 6), jnp.float32)
    return jnp.concatenate([s, ss, z], axis=1)


def _fuse_body(xyz1_ref, xyz2_ref, p1_ref, p2_ref, wfuse_ref,
               y_ref, stats_ref):
    b = pl.program_id(0)
    nb = pl.program_id(1)

    @pl.when(jnp.logical_and(b == 0, nb == 0))
    def _():
        stats_ref[...] = jnp.zeros_like(stats_ref)

    x1 = xyz1_ref[0]
    x2 = xyz2_ref[0]
    S = x2.shape[1]
    Nb = x1.shape[1]

    n1 = (x1[0:1, :] * x1[0:1, :] + x1[1:2, :] * x1[1:2, :]) \
        + x1[2:3, :] * x1[2:3, :]
    n2 = (x2[0:1, :] * x2[0:1, :] + x2[1:2, :] * x2[1:2, :]) \
        + x2[2:3, :] * x2[2:3, :]
    cross = jax.lax.dot_general(
        x2, x1, (((0,), (0,)), ((), ())),
        preferred_element_type=jnp.float32)
    d = ((-2.0 * cross) + n1) + n2.reshape(S, 1)

    iota = jax.lax.broadcasted_iota(jnp.int32, (S, Nb), 0)
    big = jnp.float32(3.0e38)
    work = d
    ams = []
    for j in range(3):
        mn = jnp.min(work, axis=0, keepdims=True)
        sel = work <= mn
        am = jnp.min(jnp.where(sel, iota, S), axis=0, keepdims=True)
        ams.append(am)
        if j < 2:
            work = jnp.where(iota == am, big, work)

    bx2 = x2.astype(jnp.bfloat16).astype(jnp.float32)
    h0 = n2.astype(jnp.bfloat16).astype(jnp.float32)
    rr = n2 - h0
    h1 = rr.astype(jnp.bfloat16).astype(jnp.float32)
    rr = rr - h1
    h2 = rr.astype(jnp.bfloat16).astype(jnp.float32)
    h3 = (rr - h2).astype(jnp.bfloat16).astype(jnp.float32)
    gmat = jnp.concatenate(
        [bx2, h0, h1, h2, h3, jnp.zeros((1, S), jnp.float32)], axis=0)

    bx1 = x1.astype(jnp.bfloat16).astype(jnp.float32)
    one = jnp.float32(1.0)
    zero = jnp.float32(0.0)
    vals = []
    for j in range(3):
        oh = jnp.where(iota == ams[j], one, zero)
        g = jax.lax.dot_general(
            gmat, oh, (((1,), (0,)), ((), ())),
            preferred_element_type=jnp.float32)
        p0 = bx1[0:1, :] * g[0:1, :]
        p1 = bx1[1:2, :] * g[1:2, :]
        p2 = bx1[2:3, :] * g[2:3, :]
        s1 = p0 + p1
        bv = s1 - p0
        e1 = (p0 - (s1 - bv)) + (p1 - bv)
        s2 = s1 + p2
        bv2 = s2 - s1
        e2 = (s1 - (s2 - bv2)) + (p2 - bv2)
        mm = s2 + (e1 + e2)
        n2sel = ((g[3:4, :] + g[4:5, :]) + g[5:6, :]) + g[6:7, :]
        vals.append(((-2.0 * mm) + n1) + n2sel)

    r0 = 1.0 / (vals[0] + 1e-8)
    r1 = 1.0 / (vals[1] + 1e-8)
    r2 = 1.0 / (vals[2] + 1e-8)
    norm = r0 + r1 + r2
    wsp = jnp.where(iota == ams[0], r0 / norm, zero)
    wsp = wsp + jnp.where(iota == ams[1], r1 / norm, zero)
    wsp = wsp + jnp.where(iota == ams[2], r2 / norm, zero)

    w_hi = wsp.astype(jnp.bfloat16).astype(jnp.float32)
    w_lo = wsp - w_hi
    p2b = p2_ref[0]
    interp = jax.lax.dot_general(
        p2b, w_hi, (((1,), (0,)), ((), ())),
        preferred_element_type=jnp.float32)
    interp = interp + jax.lax.dot_general(
        p2b, w_lo, (((1,), (0,)), ((), ())),
        preferred_element_type=jnp.float32)

    cat = jnp.concatenate([p1_ref[0], interp], axis=0)
    y = jax.lax.dot_general(
        wfuse_ref[...], cat, (((1,), (0,)), ((), ())),
        preferred_element_type=jnp.float32)

    y_ref[0] = y
    stats_ref[...] += _stats_update(y)


def _mlp_body(cnt, xin_ref, stats_in_ref, g_ref, b_ref, w_ref,
              out_ref, stats_out_ref):
    b = pl.program_id(0)
    nb = pl.program_id(1)

    @pl.when(jnp.logical_and(b == 0, nb == 0))
    def _():
        stats_out_ref[...] = jnp.zeros_like(stats_out_ref)

    scale, off = _bn_coeffs(stats_in_ref, g_ref, b_ref, cnt)
    x = _gelu(xin_ref[0] * scale + off)
    t = jax.lax.dot_general(
        w_ref[...], x, (((1,), (0,)), ((), ())),
        preferred_element_type=jnp.float32)
    out_ref[0] = t
    stats_out_ref[...] += _stats_update(t)


def _final_body(cnt, y_ref, stats_y_ref, gf_ref, bf_ref,
                t2_ref, stats2_ref, g2_ref, b2_ref, out_ref):
    scale_f, off_f = _bn_coeffs(stats_y_ref, gf_ref, bf_ref, cnt)
    x = _gelu(y_ref[0] * scale_f + off_f)
    scale2, off2 = _bn_coeffs(stats2_ref, g2_ref, b2_ref, cnt)
    h = t2_ref[0] * scale2 + off2
    out_ref[0] = _gelu(h + x)


def kernel(xyz1, xyz2, points1, points2, W_fuse, g_fuse, b_fuse,
           W1, g1, b1, W2, g2, b2):
    B, N, _ = xyz1.shape
    S = xyz2.shape[1]
    D = points1.shape[1]
    C = W_fuse.shape[0]
    cnt = float(B * N)

    Nb = 512 if N % 512 == 0 else N
    NB = N // Nb

    xyz1t = jnp.transpose(xyz1, (0, 2, 1))
    xyz2t = jnp.transpose(xyz2, (0, 2, 1))
    col = lambda v: v.reshape(C, 1)

    f32 = jnp.float32
    grid = (B, NB)

    blk_n = pl.BlockSpec((1, C, Nb), lambda b, nb: (b, 0, nb))
    blk_full = lambda shape: pl.BlockSpec(shape, lambda b, nb: (0,) * len(shape))
    blk_stats = pl.BlockSpec((C, 8), lambda b, nb: (0, 0))

    y, stats_y = pl.pallas_call(
        _fuse_body,
        grid=grid,
        in_specs=[
            pl.BlockSpec((1, 3, Nb), lambda b, nb: (b, 0, nb)),
            pl.BlockSpec((1, 3, S), lambda b, nb: (b, 0, 0)),
            blk_n,
            pl.BlockSpec((1, D, S), lambda b, nb: (b, 0, 0)),
            blk_full((C, 2 * D)),
        ],
        out_specs=[blk_n, blk_stats],
        out_shape=[
            jax.ShapeDtypeStruct((B, C, N), f32),
            jax.ShapeDtypeStruct((C, 8), f32),
        ],
    )(xyz1t, xyz2t, points1, points2, W_fuse)

    mlp = functools.partial(_mlp_body, cnt)
    mlp_call = lambda xin, stats, g, b, w: pl.pallas_call(
        mlp,
        grid=grid,
        in_specs=[
            blk_n,
            blk_stats,
            blk_full((C, 1)),
            blk_full((C, 1)),
            blk_full((C, C)),
        ],
        out_specs=[blk_n, blk_stats],
        out_shape=[
            jax.ShapeDtypeStruct((B, C, N), f32),
            jax.ShapeDtypeStruct((C, 8), f32),
        ],
    )(xin, stats, col(g), col(b), w)

    t1, stats_1 = mlp_call(y, stats_y, g_fuse, b_fuse, W1)
    t2, stats_2 = mlp_call(t1, stats_1, g1, b1, W2)

    out = pl.pallas_call(
        functools.partial(_final_body, cnt),
        grid=grid,
        in_specs=[
            blk_n,
            blk_stats,
            blk_full((C, 1)),
            blk_full((C, 1)),
            blk_n,
            blk_stats,
            blk_full((C, 1)),
            blk_full((C, 1)),
        ],
        out_specs=blk_n,
        out_shape=jax.ShapeDtypeStruct((B, C, N), f32),
    )(y, stats_y, col(g_fuse), col(b_fuse), t2, stats_2, col(g2), col(b2))

    return out


# speedup vs baseline: 23.1880x; 1.1644x over previous
import functools
import math

import jax
import jax.numpy as jnp
from jax.experimental import pallas as pl
from jax.experimental.pallas import tpu as pltpu

_INV_SQRT2 = 1.0 / math.sqrt(2.0)


def _gelu(x):
    return 0.5 * x * (1.0 + jax.lax.erf(x * _INV_SQRT2))


def _bn_coeffs(stats_ref, g_ref, b_ref, cnt):
    mean = stats_ref[:, 0:1] / cnt
    var = stats_ref[:, 1:2] / cnt - mean * mean
    scale = g_ref[...] * jax.lax.rsqrt(var + 1e-5)
    off = b_ref[...] - mean * scale
    return scale, off


def _stats_update(t):
    s = jnp.sum(t, axis=1, keepdims=True)
    ss = jnp.sum(t * t, axis=1, keepdims=True)
    z = jnp.zeros((t.shape[0], 6), jnp.float32)
    return jnp.concatenate([s, ss, z], axis=1)


def _fuse_body(xyz1_ref, xyz2_ref, p1_ref, p2_ref, wfuse_ref,
               y_ref, stats_ref):
    b = pl.program_id(0)
    nb = pl.program_id(1)

    @pl.when(jnp.logical_and(b == 0, nb == 0))
    def _():
        stats_ref[...] = jnp.zeros_like(stats_ref)

    x1 = xyz1_ref[0]
    x2 = xyz2_ref[0]
    S = x2.shape[1]
    Nb = x1.shape[1]

    n1 = (x1[0:1, :] * x1[0:1, :] + x1[1:2, :] * x1[1:2, :]) \
        + x1[2:3, :] * x1[2:3, :]
    n2 = (x2[0:1, :] * x2[0:1, :] + x2[1:2, :] * x2[1:2, :]) \
        + x2[2:3, :] * x2[2:3, :]
    cross = jax.lax.dot_general(
        x2, x1, (((0,), (0,)), ((), ())),
        preferred_element_type=jnp.float32)
    u = cross - (0.5 * n2).reshape(S, 1)

    iota = jax.lax.broadcasted_iota(jnp.int32, (S, Nb), 0)
    big = jnp.float32(3.0e38)
    work = u
    ams = []
    for j in range(3):
        mx = jnp.max(work, axis=0, keepdims=True)
        sel = work >= mx
        am = jnp.min(jnp.where(sel, iota, S), axis=0, keepdims=True)
        ams.append(am)
        if j < 2:
            work = jnp.where(iota == am, -big, work)

    bx2 = x2.astype(jnp.bfloat16).astype(jnp.float32)
    h0 = n2.astype(jnp.bfloat16).astype(jnp.float32)
    rr = n2 - h0
    h1 = rr.astype(jnp.bfloat16).astype(jnp.float32)
    rr = rr - h1
    h2 = rr.astype(jnp.bfloat16).astype(jnp.float32)
    h3 = (rr - h2).astype(jnp.bfloat16).astype(jnp.float32)
    gmat = jnp.concatenate(
        [bx2, h0, h1, h2, h3, jnp.zeros((1, S), jnp.float32)], axis=0)

    bx1 = x1.astype(jnp.bfloat16).astype(jnp.float32)
    one = jnp.float32(1.0)
    zero = jnp.float32(0.0)
    vals = []
    ohs = []
    for j in range(3):
        oh = jnp.where(iota == ams[j], one, zero)
        ohs.append(oh)
        g = jax.lax.dot_general(
            gmat, oh, (((1,), (0,)), ((), ())),
            preferred_element_type=jnp.float32)
        p0 = bx1[0:1, :] * g[0:1, :]
        p1 = bx1[1:2, :] * g[1:2, :]
        p2 = bx1[2:3, :] * g[2:3, :]
        s1 = p0 + p1
        bv = s1 - p0
        e1 = (p0 - (s1 - bv)) + (p1 - bv)
        s2 = s1 + p2
        bv2 = s2 - s1
        e2 = (s1 - (s2 - bv2)) + (p2 - bv2)
        mm = s2 + (e1 + e2)
        n2sel = ((g[3:4, :] + g[4:5, :]) + g[5:6, :]) + g[6:7, :]
        vals.append(((-2.0 * mm) + n1) + n2sel)

    r0 = 1.0 / (vals[0] + 1e-8)
    r1 = 1.0 / (vals[1] + 1e-8)
    r2 = 1.0 / (vals[2] + 1e-8)
    norm = r0 + r1 + r2
    wsp = ohs[0] * (r0 / norm)
    wsp = wsp + ohs[1] * (r1 / norm)
    wsp = wsp + ohs[2] * (r2 / norm)

    w_hi = wsp.astype(jnp.bfloat16).astype(jnp.float32)
    w_lo = wsp - w_hi
    p2b = p2_ref[0]
    interp = jax.lax.dot_general(
        p2b, w_hi, (((1,), (0,)), ((), ())),
        preferred_element_type=jnp.float32)
    interp = interp + jax.lax.dot_general(
        p2b, w_lo, (((1,), (0,)), ((), ())),
        preferred_element_type=jnp.float32)

    cat = jnp.concatenate([p1_ref[0], interp], axis=0)
    y = jax.lax.dot_general(
        wfuse_ref[...], cat, (((1,), (0,)), ((), ())),
        preferred_element_type=jnp.float32)

    y_ref[0] = y
    stats_ref[...] += _stats_update(y)


def _mlp_body(cnt, xin_ref, stats_in_ref, g_ref, b_ref, w_ref,
              out_ref, stats_out_ref):
    b = pl.program_id(0)
    nb = pl.program_id(1)

    @pl.when(jnp.logical_and(b == 0, nb == 0))
    def _():
        stats_out_ref[...] = jnp.zeros_like(stats_out_ref)

    scale, off = _bn_coeffs(stats_in_ref, g_ref, b_ref, cnt)
    x = _gelu(xin_ref[0] * scale + off)
    t = jax.lax.dot_general(
        w_ref[...], x, (((1,), (0,)), ((), ())),
        preferred_element_type=jnp.float32)
    out_ref[0] = t
    stats_out_ref[...] += _stats_update(t)


def _final_body(cnt, y_ref, stats_y_ref, gf_ref, bf_ref,
                t2_ref, stats2_ref, g2_ref, b2_ref, out_ref):
    scale_f, off_f = _bn_coeffs(stats_y_ref, gf_ref, bf_ref, cnt)
    x = _gelu(y_ref[0] * scale_f + off_f)
    scale2, off2 = _bn_coeffs(stats2_ref, g2_ref, b2_ref, cnt)
    h = t2_ref[0] * scale2 + off2
    out_ref[0] = _gelu(h + x)


def kernel(xyz1, xyz2, points1, points2, W_fuse, g_fuse, b_fuse,
           W1, g1, b1, W2, g2, b2):
    B, N, _ = xyz1.shape
    S = xyz2.shape[1]
    D = points1.shape[1]
    C = W_fuse.shape[0]
    cnt = float(B * N)

    Nb = 512 if N % 512 == 0 else N
    NB = N // Nb

    xyz1t = jnp.transpose(xyz1, (0, 2, 1))
    xyz2t = jnp.transpose(xyz2, (0, 2, 1))
    col = lambda v: v.reshape(C, 1)

    f32 = jnp.float32
    grid = (B, NB)

    blk_n = pl.BlockSpec((1, C, Nb), lambda b, nb: (b, 0, nb))
    blk_full = lambda shape: pl.BlockSpec(shape, lambda b, nb: (0,) * len(shape))
    blk_stats = pl.BlockSpec((C, 8), lambda b, nb: (0, 0))

    y, stats_y = pl.pallas_call(
        _fuse_body,
        grid=grid,
        in_specs=[
            pl.BlockSpec((1, 3, Nb), lambda b, nb: (b, 0, nb)),
            pl.BlockSpec((1, 3, S), lambda b, nb: (b, 0, 0)),
            blk_n,
            pl.BlockSpec((1, D, S), lambda b, nb: (b, 0, 0)),
            blk_full((C, 2 * D)),
        ],
        out_specs=[blk_n, blk_stats],
        out_shape=[
            jax.ShapeDtypeStruct((B, C, N), f32),
            jax.ShapeDtypeStruct((C, 8), f32),
        ],
    )(xyz1t, xyz2t, points1, points2, W_fuse)

    Nb2 = 2048 if N % 2048 == 0 else Nb
    grid2 = (B, N // Nb2)
    blk_n2 = pl.BlockSpec((1, C, Nb2), lambda b, nb: (b, 0, nb))

    mlp = functools.partial(_mlp_body, cnt)
    mlp_call = lambda xin, stats, g, b, w: pl.pallas_call(
        mlp,
        grid=grid2,
        in_specs=[
            blk_n2,
            blk_stats,
            blk_full((C, 1)),
            blk_full((C, 1)),
            blk_full((C, C)),
        ],
        out_specs=[blk_n2, blk_stats],
        out_shape=[
            jax.ShapeDtypeStruct((B, C, N), f32),
            jax.ShapeDtypeStruct((C, 8), f32),
        ],
    )(xin, stats, col(g), col(b), w)

    t1, stats_1 = mlp_call(y, stats_y, g_fuse, b_fuse, W1)
    t2, stats_2 = mlp_call(t1, stats_1, g1, b1, W2)

    out = pl.pallas_call(
        functools.partial(_final_body, cnt),
        grid=grid2,
        in_specs=[
            blk_n2,
            blk_stats,
            blk_full((C, 1)),
            blk_full((C, 1)),
            blk_n2,
            blk_stats,
            blk_full((C, 1)),
            blk_full((C, 1)),
        ],
        out_specs=blk_n2,
        out_shape=jax.ShapeDtypeStruct((B, C, N), f32),
    )(y, stats_y, col(g_fuse), col(b_fuse), t2, stats_2, col(g2), col(b2))

    return out
